# baseline (speedup 1.0000x reference)
import math
import functools

import jax
import jax.numpy as jnp
from jax import lax
from jax.experimental import pallas as pl
from jax.experimental.pallas import tpu as pltpu
from jax.experimental.pallas import tpu_sc as plsc

N0 = 10000
E = 320000
IN_DIM = 128
HID = 128
COND_DIM = 128
POS_DIM = 3
DEPTH = 3
RATIO = 0.5
TOTAL_BLOCKS = 2 * DEPTH + 1
WIDTH = max(512, 2 * HID)

BLK = 512

NC = 2
NS = 16
NW = NC * NS
LANES = 16

_MESH = plsc.VectorSubcoreMesh(core_axis_name="c", subcore_axis_name="s")


def _ceil_div(a, b):
    return (a + b - 1) // b


GM_C = 80


def _gmul(x, cols, w):
    n = x.shape[0]
    e_per_w = E // NW
    n_chunks = e_per_w // GM_C
    staged = n <= 5000

    @functools.partial(
        pl.kernel,
        mesh=_MESH,
        out_type=jax.ShapeDtypeStruct((E, HID), jnp.float32),
        scratch_types=[
            pltpu.VMEM_SHARED((n, HID), jnp.float32) if staged else None,
            pltpu.VMEM((GM_C,), jnp.int32),
            pltpu.VMEM((GM_C,), jnp.int32),
            pltpu.VMEM((GM_C,), jnp.float32),
            pltpu.VMEM((GM_C,), jnp.float32),
            pltpu.VMEM((GM_C, HID), jnp.float32),
            pltpu.VMEM((GM_C, HID), jnp.float32),
            pltpu.SemaphoreType.DMA,
            pltpu.SemaphoreType.DMA,
        ],
    )
    def k(x_hbm, cols_hbm, w_hbm, out_hbm, x_st, idx0, idx1, w0, w1,
          buf0, buf1, sem0, sem1):
        sid = lax.axis_index("s")
        wid = sid * NC + lax.axis_index("c")
        base = wid * e_per_w

        if staged:
            x_s = x_st

            @pl.when(sid == 0)
            def _():
                pltpu.sync_copy(x_hbm, x_s)

            plsc.subcore_barrier()
        else:
            x_s = x_hbm

        idxs = (idx0, idx1)
        wvs = (w0, w1)
        bufs = (buf0, buf1)
        sems = (sem0, sem1)

        pltpu.sync_copy(cols_hbm.at[pl.ds(base, GM_C)], idx0)
        pltpu.sync_copy(w_hbm.at[pl.ds(base, GM_C)], w0)
        pltpu.async_copy(x_s.at[idx0], buf0, sem0)

        def chunk(ci, carry):
            for b in range(2):
                @pl.when(lax.rem(ci, 2) == b)
                def _():
                    off = base + ci * GM_C
                    @pl.when(ci + 1 < n_chunks)
                    def _():
                        noff = base + (ci + 1) * GM_C
                        pltpu.sync_copy(cols_hbm.at[pl.ds(noff, GM_C)],
                                        idxs[1 - b])
                        pltpu.sync_copy(w_hbm.at[pl.ds(noff, GM_C)],
                                        wvs[1 - b])
                        pltpu.async_copy(x_s.at[idxs[1 - b]], bufs[1 - b],
                                         sems[1 - b])
                    pltpu.make_async_copy(x_s.at[idxs[b]], bufs[b],
                                          sems[b]).wait()
                    buf = bufs[b]
                    w_v = wvs[b]
                    for t in range(GM_C):
                        w16 = w_v[pl.ds((t // LANES) * LANES, LANES)]
                        ws = w16[t % LANES]
                        for l in range(HID // LANES):
                            s16 = pl.ds(l * LANES, LANES)
                            buf[t, s16] = buf[t, s16] * ws
                    pltpu.sync_copy(buf, out_hbm.at[pl.ds(off, GM_C)])
            return carry

        lax.fori_loop(0, n_chunks, chunk, 0)

    return k(x, cols, w)


SG_C = 2000


def _subgraph_edges(rows, cols, w, mask, newid):
    n = mask.shape[0]
    e_per_w = E // NW
    n_chunks = e_per_w // SG_C

    @functools.partial(
        pl.kernel,
        mesh=_MESH,
        out_type=[
            jax.ShapeDtypeStruct((E,), jnp.int32),
            jax.ShapeDtypeStruct((E,), jnp.int32),
            jax.ShapeDtypeStruct((E,), jnp.float32),
        ],
        scratch_types=[
            pltpu.VMEM_SHARED((n,), jnp.float32),
            pltpu.VMEM_SHARED((n,), jnp.int32),
            pltpu.VMEM((SG_C,), jnp.int32),
            pltpu.VMEM((SG_C,), jnp.int32),
            pltpu.VMEM((SG_C,), jnp.float32),
            pltpu.VMEM((SG_C,), jnp.float32),
            pltpu.VMEM((SG_C,), jnp.float32),
            pltpu.VMEM((SG_C,), jnp.int32),
            pltpu.VMEM((SG_C,), jnp.int32),
            pltpu.SemaphoreType.DMA,
        ],
    )
    def k(rows_hbm, cols_hbm, w_hbm, mask_hbm, newid_hbm,
          r2_hbm, c2_hbm, w2_hbm,
          mask_s, newid_s, ri_v, ci_v, w_v, mr_v, mc_v, nr_v, nc_v, sem):
        sid = lax.axis_index("s")
        wid = sid * NC + lax.axis_index("c")
        base = wid * e_per_w

        @pl.when(sid == 0)
        def _():
            pltpu.sync_copy(mask_hbm, mask_s)
            pltpu.sync_copy(newid_hbm, newid_s)

        plsc.subcore_barrier()

        def chunk(ci, carry):
            off = base + ci * SG_C
            sl = pl.ds(off, SG_C)
            pltpu.sync_copy(rows_hbm.at[sl], ri_v)
            pltpu.sync_copy(cols_hbm.at[sl], ci_v)
            pltpu.sync_copy(w_hbm.at[sl], w_v)
            pltpu.async_copy(mask_s.at[ri_v], mr_v, sem).wait()
            pltpu.async_copy(mask_s.at[ci_v], mc_v, sem).wait()
            pltpu.async_copy(newid_s.at[ri_v], nr_v, sem).wait()
            pltpu.async_copy(newid_s.at[ci_v], nc_v, sem).wait()
            for j in range(SG_C // LANES):
                s16 = pl.ds(j * LANES, LANES)
                w_v[s16] = w_v[s16] * (mr_v[s16] * mc_v[s16])
            pltpu.sync_copy(nr_v, r2_hbm.at[sl])
            pltpu.sync_copy(nc_v, c2_hbm.at[sl])
            pltpu.sync_copy(w_v, w2_hbm.at[sl])
            return carry

        lax.fori_loop(0, n_chunks, chunk, 0)

    return k(rows, cols, w, mask, newid)


def _wnorm(rows, cols, w, dinv):
    n = dinv.shape[0]
    e_per_w = E // NW
    n_chunks = e_per_w // SG_C

    @functools.partial(
        pl.kernel,
        mesh=_MESH,
        out_type=jax.ShapeDtypeStruct((E,), jnp.float32),
        scratch_types=[
            pltpu.VMEM_SHARED((n,), jnp.float32),
            pltpu.VMEM((SG_C,), jnp.int32),
            pltpu.VMEM((SG_C,), jnp.int32),
            pltpu.VMEM((SG_C,), jnp.float32),
            pltpu.VMEM((SG_C,), jnp.float32),
            pltpu.VMEM((SG_C,), jnp.float32),
            pltpu.SemaphoreType.DMA,
        ],
    )
    def k(rows_hbm, cols_hbm, w_hbm, dinv_hbm, wh_hbm,
          dinv_s, ri_v, ci_v, w_v, dr_v, dc_v, sem):
        sid = lax.axis_index("s")
        wid = sid * NC + lax.axis_index("c")
        base = wid * e_per_w

        @pl.when(sid == 0)
        def _():
            pltpu.sync_copy(dinv_hbm, dinv_s)

        plsc.subcore_barrier()

        def chunk(ci, carry):
            off = base + ci * SG_C
            sl = pl.ds(off, SG_C)
            pltpu.sync_copy(rows_hbm.at[sl], ri_v)
            pltpu.sync_copy(cols_hbm.at[sl], ci_v)
            pltpu.sync_copy(w_hbm.at[sl], w_v)
            pltpu.async_copy(dinv_s.at[ri_v], dr_v, sem).wait()
            pltpu.async_copy(dinv_s.at[ci_v], dc_v, sem).wait()
            for j in range(SG_C // LANES):
                s16 = pl.ds(j * LANES, LANES)
                w_v[s16] = (dr_v[s16] * w_v[s16]) * dc_v[s16]
            pltpu.sync_copy(w_v, wh_hbm.at[sl])
            return carry

        lax.fori_loop(0, n_chunks, chunk, 0)

    return k(rows, cols, w, dinv)


SP_C = 80


def _spmm_fused(g, rows, cols, w):
    n = g.shape[0]
    npad = ((n + 127) // 128) * 128
    rpt = npad // NS
    e_per_w = E // NW
    n_chunks = e_per_w // SP_C
    staged = n <= 5000

    @functools.partial(
        pl.kernel,
        mesh=_MESH,
        out_type=jax.ShapeDtypeStruct((NC, npad, HID), jnp.float32),
        scratch_types=[
            pltpu.VMEM_SHARED((npad, HID), jnp.float32),
            pltpu.VMEM_SHARED((n, HID), jnp.float32) if staged else None,
            pltpu.VMEM((SP_C,), jnp.int32),
            pltpu.VMEM((SP_C,), jnp.int32),
            pltpu.VMEM((SP_C,), jnp.int32),
            pltpu.VMEM((SP_C,), jnp.int32),
            pltpu.VMEM((SP_C,), jnp.float32),
            pltpu.VMEM((SP_C,), jnp.float32),
            pltpu.VMEM((SP_C, HID), jnp.float32),
            pltpu.VMEM((SP_C, HID), jnp.float32),
            pltpu.SemaphoreType.DMA,
            pltpu.SemaphoreType.DMA,
        ],
    )
    def k(g_hbm, rows_hbm, cols_hbm, w_hbm, zeros_hbm, out_hbm,
          acc_s, g_st, ri0, ri1, ci0, ci1, w0, w1, buf0, buf1, sem0, sem1):
        cid = lax.axis_index("c")
        sid = lax.axis_index("s")
        wid = sid * NC + cid
        base = wid * e_per_w

        tsl = pl.ds(sid * rpt, rpt)
        pltpu.sync_copy(zeros_hbm.at[tsl], acc_s.at[tsl])
        if staged:
            g_s = g_st

            @pl.when(sid == 0)
            def _():
                pltpu.sync_copy(g_hbm, g_s)
        else:
            g_s = g_hbm
        plsc.subcore_barrier()

        ris = (ri0, ri1)
        cis = (ci0, ci1)
        wvs = (w0, w1)
        bufs = (buf0, buf1)
        sems = (sem0, sem1)

        pltpu.sync_copy(rows_hbm.at[pl.ds(base, SP_C)], ri0)
        pltpu.sync_copy(cols_hbm.at[pl.ds(base, SP_C)], ci0)
        pltpu.sync_copy(w_hbm.at[pl.ds(base, SP_C)], w0)
        pltpu.async_copy(g_s.at[ci0], buf0, sem0)

        def chunk(ci, carry):
            for b in range(2):
                @pl.when(lax.rem(ci, 2) == b)
                def _():
                    @pl.when(ci + 1 < n_chunks)
                    def _():
                        noff = base + (ci + 1) * SP_C
                        nsl = pl.ds(noff, SP_C)
                        pltpu.sync_copy(rows_hbm.at[nsl], ris[1 - b])
                        pltpu.sync_copy(cols_hbm.at[nsl], cis[1 - b])
                        pltpu.sync_copy(w_hbm.at[nsl], wvs[1 - b])
                        pltpu.async_copy(g_s.at[cis[1 - b]], bufs[1 - b],
                                         sems[1 - b])
                    pltpu.make_async_copy(g_s.at[cis[b]], bufs[b],
                                          sems[b]).wait()
                    buf = bufs[b]
                    w_v = wvs[b]
                    for t in range(SP_C):
                        w16 = w_v[pl.ds((t // LANES) * LANES, LANES)]
                        ws = w16[t % LANES]
                        for l in range(HID // LANES):
                            s16 = pl.ds(l * LANES, LANES)
                            buf[t, s16] = buf[t, s16] * ws
                    pltpu.sync_copy(buf, acc_s.at[ris[b]], add=True)
            return carry

        lax.fori_loop(0, n_chunks, chunk, 0)
        plsc.subcore_barrier()
        pltpu.sync_copy(acc_s.at[tsl], out_hbm.at[cid, tsl])

    zeros = jnp.zeros((npad, HID), jnp.float32)
    out = k(g, rows, cols, w, zeros)
    return out[0, :n, :], out[1, :n, :]


def _pre_body(x_ref, lns_ref, lnb_ref, w_ref, o_ref):
    x = x_ref[...]
    m = jnp.mean(x, axis=-1, keepdims=True)
    v = jnp.mean((x - m) ** 2, axis=-1, keepdims=True)
    h = (x - m) * jax.lax.rsqrt(v + 1e-5) * lns_ref[...] + lnb_ref[...]
    h = h * jax.nn.sigmoid(h)
    o_ref[...] = jnp.dot(h, w_ref[...], preferred_element_type=jnp.float32)


def _block_pre(x, ln, W):
    n = x.shape[0]
    lns, lnb = ln[0].reshape(1, HID), ln[1].reshape(1, HID)
    grid = (_ceil_div(n, BLK),)
    return pl.pallas_call(
        _pre_body,
        grid=grid,
        in_specs=[
            pl.BlockSpec((BLK, HID), lambda i: (i, 0)),
            pl.BlockSpec((1, HID), lambda i: (0, 0)),
            pl.BlockSpec((1, HID), lambda i: (0, 0)),
            pl.BlockSpec((HID, HID), lambda i: (0, 0)),
        ],
        out_specs=pl.BlockSpec((BLK, HID), lambda i: (i, 0)),
        out_shape=jax.ShapeDtypeStruct((n, HID), jnp.float32),
    )(x, lns, lnb, W)


def _post_body(x_ref, p0_ref, p1_ref, g_ref, diag_ref, b_ref, gm_ref, bt_ref,
               o_ref):
    h = (p0_ref[...] + p1_ref[...]) + diag_ref[...] * g_ref[...] + b_ref[...]
    h = h * gm_ref[...] + bt_ref[...]
    o_ref[...] = x_ref[...] + h


def _block_post(x, p0, p1, g, diag, b, gamma, beta):
    n = x.shape[0]
    grid = (_ceil_div(n, BLK),)
    row_spec = pl.BlockSpec((BLK, HID), lambda i: (i, 0))
    vec_spec = pl.BlockSpec((1, HID), lambda i: (0, 0))
    return pl.pallas_call(
        _post_body,
        grid=grid,
        in_specs=[
            row_spec,
            row_spec,
            row_spec,
            row_spec,
            pl.BlockSpec((BLK, 1), lambda i: (i, 0)),
            vec_spec,
            vec_spec,
            vec_spec,
        ],
        out_specs=row_spec,
        out_shape=jax.ShapeDtypeStruct((n, HID), jnp.float32),
    )(x, p0, p1, g, diag.reshape(n, 1), b.reshape(1, HID),
      gamma.reshape(1, HID), beta.reshape(1, HID))


def _outproj_body(x_ref, lns_ref, lnb_ref, w_ref, b_ref, o_ref):
    x = x_ref[...]
    m = jnp.mean(x, axis=-1, keepdims=True)
    v = jnp.mean((x - m) ** 2, axis=-1, keepdims=True)
    h = (x - m) * jax.lax.rsqrt(v + 1e-5) * lns_ref[...] + lnb_ref[...]
    o_ref[...] = jnp.dot(h, w_ref[...], preferred_element_type=jnp.float32) + b_ref[...]


def _out_proj(x, ln, W, b):
    n = x.shape[0]
    grid = (_ceil_div(n, BLK),)
    return pl.pallas_call(
        _outproj_body,
        grid=grid,
        in_specs=[
            pl.BlockSpec((BLK, HID), lambda i: (i, 0)),
            pl.BlockSpec((1, HID), lambda i: (0, 0)),
            pl.BlockSpec((1, HID), lambda i: (0, 0)),
            pl.BlockSpec((HID, IN_DIM), lambda i: (0, 0)),
            pl.BlockSpec((1, IN_DIM), lambda i: (0, 0)),
        ],
        out_specs=pl.BlockSpec((BLK, IN_DIM), lambda i: (i, 0)),
        out_shape=jax.ShapeDtypeStruct((n, IN_DIM), jnp.float32),
    )(x, ln[0].reshape(1, HID), ln[1].reshape(1, HID), W, b.reshape(1, IN_DIM))


def _layernorm(x, p, eps=1e-5):
    s, b = p
    m = jnp.mean(x, axis=-1, keepdims=True)
    v = jnp.mean((x - m) ** 2, axis=-1, keepdims=True)
    return (x - m) * jax.lax.rsqrt(v + eps) * s + b


def _mlp2(x, p):
    (W1, b1), (W2, b2) = p
    return jax.nn.silu(x @ W1 + b1) @ W2 + b2


def _film(cond, p):
    (W1, b1), (W2, b2), (W3, b3) = p
    h = jax.nn.silu(cond @ W1 + b1)
    h = jax.nn.silu(h @ W2 + b2)
    f = (h @ W3 + b3).reshape(TOTAL_BLOCKS, 2, HID)
    return 1.0 + f[:, 0, :], f[:, 1, :]


def _gcn_norm_sc(rows, cols, w, diag, n):
    diag = diag + 1.0
    deg = jax.ops.segment_sum(w, rows, num_segments=n) + diag
    dinv = jax.lax.rsqrt(deg)
    return _wnorm(rows, cols, w, dinv), diag * dinv * dinv


def _spmm_seg(adj, g):
    rows, cols, w, diag, n = adj
    return jax.ops.segment_sum(_gmul(g, cols, w), rows, num_segments=n)


def _block_xla(x, adj, p, gamma, beta):
    rows, cols, w, diag, n = adj
    W, b = p["lin"]
    h = jax.nn.silu(_layernorm(x, p["ln"]))
    h = (_spmm_seg(adj, h) + diag[:, None] * h) @ W + b
    h = h * gamma[None, :] + beta[None, :]
    return x + h


def _pool(x, p):
    n = x.shape[0]
    k = max(1, int(math.ceil(RATIO * n)))
    W1, b1 = p["s1"]
    W2, b2 = p["s2"]
    s = (jax.nn.silu(_layernorm(x, p["ln"]) @ W1 + b1) @ W2 + b2)[:, 0]
    _, keep = jax.lax.top_k(s, k)
    return x[keep], keep


def _subgraph_sc(adj, keep, k):
    rows, cols, w, diag, n = adj
    mask = jnp.zeros((n,), w.dtype).at[keep].set(1.0)
    newid = jnp.zeros((n,), rows.dtype).at[keep].set(jnp.arange(k, dtype=rows.dtype))
    rows2, cols2, w2 = _subgraph_edges(rows, cols, w, mask, newid)
    return (rows2, cols2, w2, diag[keep], k)


def _block_p2(x, adj, p, gamma, beta):
    rows, cols, w, diag, n = adj
    W, b = p["lin"]
    g = _block_pre(x, p["ln"], W)
    p0, p1 = _spmm_fused(g, rows, cols, w)
    return _block_post(x, p0, p1, g, diag, b, gamma, beta)


def kernel(x0, edge_index, cond, pos, params):
    rows, cols = edge_index[0], edge_index[1]
    n = x0.shape[0]

    w0 = jnp.ones((rows.shape[0],), x0.dtype)
    d0 = jnp.zeros((n,), x0.dtype)
    wh, dh = _gcn_norm_sc(rows, cols, w0, d0, n)
    adj = (rows, cols, wh, dh, n)
    gammas, betas = _film(cond, params["film"])
    Wi, bi = params["in_proj"]
    h = x0 @ Wi + bi + _mlp2(pos, params["pos_mlp"])
    g = 0
    skips = []
    adjs = []
    for d in range(DEPTH):
        for p in params["enc"][d]:
            h = _block_xla(h, adj, p, gammas[g], betas[g])
            g += 1
        h_skip = h
        h_pool, keep = _pool(h, params["pools"][d])
        k = h_pool.shape[0]
        sub = _subgraph_sc(adj, keep, k)
        w2h, dg2h = _gcn_norm_sc(sub[0], sub[1], sub[2], sub[3], k)
        skips.append((h_skip, keep, adj[4]))
        adjs.append(adj)
        adj = (sub[0], sub[1], w2h, dg2h, k)
        h = h_pool

    for p in params["bottleneck"]:
        h = _block_p2(h, adj, p, gammas[g], betas[g])
        g += 1
    for d in reversed(range(DEPTH)):
        h_skip, keep, n_prev = skips[d]
        h = jnp.zeros((n_prev, HID), h.dtype).at[keep].set(h) + h_skip
        for p in params["dec"][d]:
            h = _block_p2(h, adjs[d], p, gammas[g], betas[g])
            g += 1
    Wo, bo = params["out_proj"]
    return _out_proj(h, params["out_norm"], Wo, bo)


# speedup vs baseline: 1.0432x; 1.0432x over previous
import math
import functools

import jax
import jax.numpy as jnp
from jax import lax
from jax.experimental import pallas as pl
from jax.experimental.pallas import tpu as pltpu
from jax.experimental.pallas import tpu_sc as plsc

N0 = 10000
E = 320000
IN_DIM = 128
HID = 128
COND_DIM = 128
POS_DIM = 3
DEPTH = 3
RATIO = 0.5
TOTAL_BLOCKS = 2 * DEPTH + 1
WIDTH = max(512, 2 * HID)

BLK = 512

NC = 2
NS = 16
NW = NC * NS
LANES = 16

_MESH = plsc.VectorSubcoreMesh(core_axis_name="c", subcore_axis_name="s")


def _ceil_div(a, b):
    return (a + b - 1) // b


GM_C = 200


def _grows(x, cols):
    n = x.shape[0]
    e_per_w = E // NW
    n_chunks = e_per_w // GM_C
    staged = n <= 5000

    @functools.partial(
        pl.kernel,
        mesh=_MESH,
        out_type=jax.ShapeDtypeStruct((E, HID), jnp.float32),
        scratch_types=[
            pltpu.VMEM_SHARED((n, HID), jnp.float32) if staged else None,
            pltpu.VMEM((GM_C,), jnp.int32),
            pltpu.VMEM((GM_C,), jnp.int32),
            pltpu.VMEM((GM_C, HID), jnp.float32),
            pltpu.VMEM((GM_C, HID), jnp.float32),
            pltpu.SemaphoreType.DMA,
            pltpu.SemaphoreType.DMA,
        ],
    )
    def k(x_hbm, cols_hbm, out_hbm, x_st, idx0, idx1, buf0, buf1, sem0, sem1):
        sid = lax.axis_index("s")
        wid = sid * NC + lax.axis_index("c")
        base = wid * e_per_w

        if staged:
            x_s = x_st

            @pl.when(sid == 0)
            def _():
                pltpu.sync_copy(x_hbm, x_s)

            plsc.subcore_barrier()
        else:
            x_s = x_hbm

        idxs = (idx0, idx1)
        bufs = (buf0, buf1)
        sems = (sem0, sem1)

        pltpu.sync_copy(cols_hbm.at[pl.ds(base, GM_C)], idx0)
        pltpu.async_copy(x_s.at[idx0], buf0, sem0)

        def chunk(ci, carry):
            for b in range(2):
                @pl.when(lax.rem(ci, 2) == b)
                def _():
                    off = base + ci * GM_C
                    @pl.when(ci + 1 < n_chunks)
                    def _():
                        noff = base + (ci + 1) * GM_C
                        pltpu.sync_copy(cols_hbm.at[pl.ds(noff, GM_C)],
                                        idxs[1 - b])
                        pltpu.async_copy(x_s.at[idxs[1 - b]], bufs[1 - b],
                                         sems[1 - b])
                    pltpu.make_async_copy(x_s.at[idxs[b]], bufs[b],
                                          sems[b]).wait()
                    pltpu.sync_copy(bufs[b], out_hbm.at[pl.ds(off, GM_C)])
            return carry

        lax.fori_loop(0, n_chunks, chunk, 0)

    return k(x, cols)


SG_C = 2000


def _subgraph_edges(rows, cols, w, mask, newid):
    n = mask.shape[0]
    e_per_w = E // NW
    n_chunks = e_per_w // SG_C

    @functools.partial(
        pl.kernel,
        mesh=_MESH,
        out_type=[
            jax.ShapeDtypeStruct((E,), jnp.int32),
            jax.ShapeDtypeStruct((E,), jnp.int32),
            jax.ShapeDtypeStruct((E,), jnp.float32),
        ],
        scratch_types=[
            pltpu.VMEM_SHARED((n,), jnp.float32),
            pltpu.VMEM_SHARED((n,), jnp.int32),
            pltpu.VMEM((SG_C,), jnp.int32),
            pltpu.VMEM((SG_C,), jnp.int32),
            pltpu.VMEM((SG_C,), jnp.float32),
            pltpu.VMEM((SG_C,), jnp.float32),
            pltpu.VMEM((SG_C,), jnp.float32),
            pltpu.VMEM((SG_C,), jnp.int32),
            pltpu.VMEM((SG_C,), jnp.int32),
            pltpu.SemaphoreType.DMA,
        ],
    )
    def k(rows_hbm, cols_hbm, w_hbm, mask_hbm, newid_hbm,
          r2_hbm, c2_hbm, w2_hbm,
          mask_s, newid_s, ri_v, ci_v, w_v, mr_v, mc_v, nr_v, nc_v, sem):
        sid = lax.axis_index("s")
        wid = sid * NC + lax.axis_index("c")
        base = wid * e_per_w

        @pl.when(sid == 0)
        def _():
            pltpu.sync_copy(mask_hbm, mask_s)
            pltpu.sync_copy(newid_hbm, newid_s)

        plsc.subcore_barrier()

        def chunk(ci, carry):
            off = base + ci * SG_C
            sl = pl.ds(off, SG_C)
            pltpu.sync_copy(rows_hbm.at[sl], ri_v)
            pltpu.sync_copy(cols_hbm.at[sl], ci_v)
            pltpu.sync_copy(w_hbm.at[sl], w_v)
            pltpu.async_copy(mask_s.at[ri_v], mr_v, sem).wait()
            pltpu.async_copy(mask_s.at[ci_v], mc_v, sem).wait()
            pltpu.async_copy(newid_s.at[ri_v], nr_v, sem).wait()
            pltpu.async_copy(newid_s.at[ci_v], nc_v, sem).wait()
            for j in range(SG_C // LANES):
                s16 = pl.ds(j * LANES, LANES)
                w_v[s16] = w_v[s16] * (mr_v[s16] * mc_v[s16])
            pltpu.sync_copy(nr_v, r2_hbm.at[sl])
            pltpu.sync_copy(nc_v, c2_hbm.at[sl])
            pltpu.sync_copy(w_v, w2_hbm.at[sl])
            return carry

        lax.fori_loop(0, n_chunks, chunk, 0)

    return k(rows, cols, w, mask, newid)


def _wnorm(rows, cols, w, dinv):
    n = dinv.shape[0]
    e_per_w = E // NW
    n_chunks = e_per_w // SG_C

    @functools.partial(
        pl.kernel,
        mesh=_MESH,
        out_type=jax.ShapeDtypeStruct((E,), jnp.float32),
        scratch_types=[
            pltpu.VMEM_SHARED((n,), jnp.float32),
            pltpu.VMEM((SG_C,), jnp.int32),
            pltpu.VMEM((SG_C,), jnp.int32),
            pltpu.VMEM((SG_C,), jnp.float32),
            pltpu.VMEM((SG_C,), jnp.float32),
            pltpu.VMEM((SG_C,), jnp.float32),
            pltpu.SemaphoreType.DMA,
        ],
    )
    def k(rows_hbm, cols_hbm, w_hbm, dinv_hbm, wh_hbm,
          dinv_s, ri_v, ci_v, w_v, dr_v, dc_v, sem):
        sid = lax.axis_index("s")
        wid = sid * NC + lax.axis_index("c")
        base = wid * e_per_w

        @pl.when(sid == 0)
        def _():
            pltpu.sync_copy(dinv_hbm, dinv_s)

        plsc.subcore_barrier()

        def chunk(ci, carry):
            off = base + ci * SG_C
            sl = pl.ds(off, SG_C)
            pltpu.sync_copy(rows_hbm.at[sl], ri_v)
            pltpu.sync_copy(cols_hbm.at[sl], ci_v)
            pltpu.sync_copy(w_hbm.at[sl], w_v)
            pltpu.async_copy(dinv_s.at[ri_v], dr_v, sem).wait()
            pltpu.async_copy(dinv_s.at[ci_v], dc_v, sem).wait()
            for j in range(SG_C // LANES):
                s16 = pl.ds(j * LANES, LANES)
                w_v[s16] = (dr_v[s16] * w_v[s16]) * dc_v[s16]
            pltpu.sync_copy(w_v, wh_hbm.at[sl])
            return carry

        lax.fori_loop(0, n_chunks, chunk, 0)

    return k(rows, cols, w, dinv)


SP_C = 80


def _spmm_fused(g, rows, cols, w):
    n = g.shape[0]
    npad = ((n + 127) // 128) * 128
    rpt = npad // NS
    e_per_w = E // NW
    n_chunks = e_per_w // SP_C
    staged = n <= 5000

    @functools.partial(
        pl.kernel,
        mesh=_MESH,
        out_type=jax.ShapeDtypeStruct((NC, npad, HID), jnp.float32),
        scratch_types=[
            pltpu.VMEM_SHARED((npad, HID), jnp.float32),
            pltpu.VMEM_SHARED((n, HID), jnp.float32) if staged else None,
            pltpu.VMEM((SP_C,), jnp.int32),
            pltpu.VMEM((SP_C,), jnp.int32),
            pltpu.VMEM((SP_C,), jnp.int32),
            pltpu.VMEM((SP_C,), jnp.int32),
            pltpu.VMEM((SP_C,), jnp.float32),
            pltpu.VMEM((SP_C,), jnp.float32),
            pltpu.VMEM((SP_C, HID), jnp.float32),
            pltpu.VMEM((SP_C, HID), jnp.float32),
            pltpu.SemaphoreType.DMA,
            pltpu.SemaphoreType.DMA,
        ],
    )
    def k(g_hbm, rows_hbm, cols_hbm, w_hbm, zeros_hbm, out_hbm,
          acc_s, g_st, ri0, ri1, ci0, ci1, w0, w1, buf0, buf1, sem0, sem1):
        cid = lax.axis_index("c")
        sid = lax.axis_index("s")
        wid = sid * NC + cid
        base = wid * e_per_w

        tsl = pl.ds(sid * rpt, rpt)
        pltpu.sync_copy(zeros_hbm.at[tsl], acc_s.at[tsl])
        if staged:
            g_s = g_st

            @pl.when(sid == 0)
            def _():
                pltpu.sync_copy(g_hbm, g_s)
        else:
            g_s = g_hbm
        plsc.subcore_barrier()

        ris = (ri0, ri1)
        cis = (ci0, ci1)
        wvs = (w0, w1)
        bufs = (buf0, buf1)
        sems = (sem0, sem1)

        pltpu.sync_copy(rows_hbm.at[pl.ds(base, SP_C)], ri0)
        pltpu.sync_copy(cols_hbm.at[pl.ds(base, SP_C)], ci0)
        pltpu.sync_copy(w_hbm.at[pl.ds(base, SP_C)], w0)
        pltpu.async_copy(g_s.at[ci0], buf0, sem0)

        def chunk(ci, carry):
            for b in range(2):
                @pl.when(lax.rem(ci, 2) == b)
                def _():
                    @pl.when(ci + 1 < n_chunks)
                    def _():
                        noff = base + (ci + 1) * SP_C
                        nsl = pl.ds(noff, SP_C)
                        pltpu.sync_copy(rows_hbm.at[nsl], ris[1 - b])
                        pltpu.sync_copy(cols_hbm.at[nsl], cis[1 - b])
                        pltpu.sync_copy(w_hbm.at[nsl], wvs[1 - b])
                        pltpu.async_copy(g_s.at[cis[1 - b]], bufs[1 - b],
                                         sems[1 - b])
                    pltpu.make_async_copy(g_s.at[cis[b]], bufs[b],
                                          sems[b]).wait()
                    buf = bufs[b]
                    w_v = wvs[b]
                    for t in range(SP_C):
                        w16 = w_v[pl.ds((t // LANES) * LANES, LANES)]
                        ws = w16[t % LANES]
                        for l in range(HID // LANES):
                            s16 = pl.ds(l * LANES, LANES)
                            buf[t, s16] = buf[t, s16] * ws
                    pltpu.sync_copy(buf, acc_s.at[ris[b]], add=True)
            return carry

        lax.fori_loop(0, n_chunks, chunk, 0)
        plsc.subcore_barrier()
        pltpu.sync_copy(acc_s.at[tsl], out_hbm.at[cid, tsl])

    zeros = jnp.zeros((npad, HID), jnp.float32)
    out = k(g, rows, cols, w, zeros)
    return out[0, :n, :], out[1, :n, :]


def _pre_body(x_ref, lns_ref, lnb_ref, w_ref, o_ref):
    x = x_ref[...]
    m = jnp.mean(x, axis=-1, keepdims=True)
    v = jnp.mean((x - m) ** 2, axis=-1, keepdims=True)
    h = (x - m) * jax.lax.rsqrt(v + 1e-5) * lns_ref[...] + lnb_ref[...]
    h = h * jax.nn.sigmoid(h)
    o_ref[...] = jnp.dot(h, w_ref[...], preferred_element_type=jnp.float32)


def _block_pre(x, ln, W):
    n = x.shape[0]
    lns, lnb = ln[0].reshape(1, HID), ln[1].reshape(1, HID)
    grid = (_ceil_div(n, BLK),)
    return pl.pallas_call(
        _pre_body,
        grid=grid,
        in_specs=[
            pl.BlockSpec((BLK, HID), lambda i: (i, 0)),
            pl.BlockSpec((1, HID), lambda i: (0, 0)),
            pl.BlockSpec((1, HID), lambda i: (0, 0)),
            pl.BlockSpec((HID, HID), lambda i: (0, 0)),
        ],
        out_specs=pl.BlockSpec((BLK, HID), lambda i: (i, 0)),
        out_shape=jax.ShapeDtypeStruct((n, HID), jnp.float32),
    )(x, lns, lnb, W)


def _post_body(x_ref, p0_ref, p1_ref, g_ref, diag_ref, b_ref, gm_ref, bt_ref,
               o_ref):
    h = (p0_ref[...] + p1_ref[...]) + diag_ref[...] * g_ref[...] + b_ref[...]
    h = h * gm_ref[...] + bt_ref[...]
    o_ref[...] = x_ref[...] + h


def _block_post(x, p0, p1, g, diag, b, gamma, beta):
    n = x.shape[0]
    grid = (_ceil_div(n, BLK),)
    row_spec = pl.BlockSpec((BLK, HID), lambda i: (i, 0))
    vec_spec = pl.BlockSpec((1, HID), lambda i: (0, 0))
    return pl.pallas_call(
        _post_body,
        grid=grid,
        in_specs=[
            row_spec,
            row_spec,
            row_spec,
            row_spec,
            pl.BlockSpec((BLK, 1), lambda i: (i, 0)),
            vec_spec,
            vec_spec,
            vec_spec,
        ],
        out_specs=row_spec,
        out_shape=jax.ShapeDtypeStruct((n, HID), jnp.float32),
    )(x, p0, p1, g, diag.reshape(n, 1), b.reshape(1, HID),
      gamma.reshape(1, HID), beta.reshape(1, HID))


def _outproj_body(x_ref, lns_ref, lnb_ref, w_ref, b_ref, o_ref):
    x = x_ref[...]
    m = jnp.mean(x, axis=-1, keepdims=True)
    v = jnp.mean((x - m) ** 2, axis=-1, keepdims=True)
    h = (x - m) * jax.lax.rsqrt(v + 1e-5) * lns_ref[...] + lnb_ref[...]
    o_ref[...] = jnp.dot(h, w_ref[...], preferred_element_type=jnp.float32) + b_ref[...]


def _out_proj(x, ln, W, b):
    n = x.shape[0]
    grid = (_ceil_div(n, BLK),)
    return pl.pallas_call(
        _outproj_body,
        grid=grid,
        in_specs=[
            pl.BlockSpec((BLK, HID), lambda i: (i, 0)),
            pl.BlockSpec((1, HID), lambda i: (0, 0)),
            pl.BlockSpec((1, HID), lambda i: (0, 0)),
            pl.BlockSpec((HID, IN_DIM), lambda i: (0, 0)),
            pl.BlockSpec((1, IN_DIM), lambda i: (0, 0)),
        ],
        out_specs=pl.BlockSpec((BLK, IN_DIM), lambda i: (i, 0)),
        out_shape=jax.ShapeDtypeStruct((n, IN_DIM), jnp.float32),
    )(x, ln[0].reshape(1, HID), ln[1].reshape(1, HID), W, b.reshape(1, IN_DIM))


def _layernorm(x, p, eps=1e-5):
    s, b = p
    m = jnp.mean(x, axis=-1, keepdims=True)
    v = jnp.mean((x - m) ** 2, axis=-1, keepdims=True)
    return (x - m) * jax.lax.rsqrt(v + eps) * s + b


def _mlp2(x, p):
    (W1, b1), (W2, b2) = p
    return jax.nn.silu(x @ W1 + b1) @ W2 + b2


def _film(cond, p):
    (W1, b1), (W2, b2), (W3, b3) = p
    h = jax.nn.silu(cond @ W1 + b1)
    h = jax.nn.silu(h @ W2 + b2)
    f = (h @ W3 + b3).reshape(TOTAL_BLOCKS, 2, HID)
    return 1.0 + f[:, 0, :], f[:, 1, :]


def _gcn_norm_sc(rows, cols, w, diag, n):
    diag = diag + 1.0
    deg = jax.ops.segment_sum(w, rows, num_segments=n) + diag
    dinv = jax.lax.rsqrt(deg)
    return _wnorm(rows, cols, w, dinv), diag * dinv * dinv


def _spmm_seg(adj, g):
    rows, cols, w, diag, n = adj
    prod = w[:, None] * _grows(g, cols)
    return jax.ops.segment_sum(prod, rows, num_segments=n)


def _block_xla(x, adj, p, gamma, beta):
    rows, cols, w, diag, n = adj
    W, b = p["lin"]
    h = jax.nn.silu(_layernorm(x, p["ln"]))
    h = (_spmm_seg(adj, h) + diag[:, None] * h) @ W + b
    h = h * gamma[None, :] + beta[None, :]
    return x + h


def _pool(x, p):
    n = x.shape[0]
    k = max(1, int(math.ceil(RATIO * n)))
    W1, b1 = p["s1"]
    W2, b2 = p["s2"]
    s = (jax.nn.silu(_layernorm(x, p["ln"]) @ W1 + b1) @ W2 + b2)[:, 0]
    _, keep = jax.lax.top_k(s, k)
    return x[keep], keep


def _subgraph_sc(adj, keep, k):
    rows, cols, w, diag, n = adj
    mask = jnp.zeros((n,), w.dtype).at[keep].set(1.0)
    newid = jnp.zeros((n,), rows.dtype).at[keep].set(jnp.arange(k, dtype=rows.dtype))
    rows2, cols2, w2 = _subgraph_edges(rows, cols, w, mask, newid)
    return (rows2, cols2, w2, diag[keep], k)


def _block_p2(x, adj, p, gamma, beta):
    rows, cols, w, diag, n = adj
    W, b = p["lin"]
    g = _block_pre(x, p["ln"], W)
    p0, p1 = _spmm_fused(g, rows, cols, w)
    return _block_post(x, p0, p1, g, diag, b, gamma, beta)


def kernel(x0, edge_index, cond, pos, params):
    rows, cols = edge_index[0], edge_index[1]
    n = x0.shape[0]

    w0 = jnp.ones((rows.shape[0],), x0.dtype)
    d0 = jnp.zeros((n,), x0.dtype)
    wh, dh = _gcn_norm_sc(rows, cols, w0, d0, n)
    adj = (rows, cols, wh, dh, n)
    gammas, betas = _film(cond, params["film"])
    Wi, bi = params["in_proj"]
    h = x0 @ Wi + bi + _mlp2(pos, params["pos_mlp"])
    g = 0
    skips = []
    adjs = []
    for d in range(DEPTH):
        for p in params["enc"][d]:
            h = _block_xla(h, adj, p, gammas[g], betas[g])
            g += 1
        h_skip = h
        h_pool, keep = _pool(h, params["pools"][d])
        k = h_pool.shape[0]
        sub = _subgraph_sc(adj, keep, k)
        w2h, dg2h = _gcn_norm_sc(sub[0], sub[1], sub[2], sub[3], k)
        skips.append((h_skip, keep, adj[4]))
        adjs.append(adj)
        adj = (sub[0], sub[1], w2h, dg2h, k)
        h = h_pool

    for p in params["bottleneck"]:
        h = _block_p2(h, adj, p, gammas[g], betas[g])
        g += 1
    for d in reversed(range(DEPTH)):
        h_skip, keep, n_prev = skips[d]
        h = jnp.zeros((n_prev, HID), h.dtype).at[keep].set(h) + h_skip
        for p in params["dec"][d]:
            h = _block_p2(h, adjs[d], p, gammas[g], betas[g])
            g += 1
    Wo, bo = params["out_proj"]
    return _out_proj(h, params["out_norm"], Wo, bo)
